# baseline (speedup 1.0000x reference)
import jax
import jax.numpy as jnp
from jax.experimental import pallas as pl

_BM = 128


def _mm_kernel(m_ref, x_ref, o_ref):
    m = m_ref[...].astype(jnp.bfloat16)
    xv = x_ref[...].astype(jnp.bfloat16)
    o_ref[...] = jax.lax.dot_general(
        m, xv, (((1,), (0,)), ((), ())),
        preferred_element_type=jnp.float32)


def kernel(x, M):
    B, C, V = x.shape
    Vo = M.shape[0]
    N = B * C
    xr = jnp.transpose(x, (2, 0, 1)).reshape(V, N)
    out = pl.pallas_call(
        _mm_kernel,
        grid=(pl.cdiv(Vo, _BM),),
        in_specs=[
            pl.BlockSpec((_BM, V), lambda i: (i, 0)),
            pl.BlockSpec((V, N), lambda i: (0, 0)),
        ],
        out_specs=pl.BlockSpec((_BM, N), lambda i: (i, 0)),
        out_shape=jax.ShapeDtypeStruct((Vo, N), jnp.float32),
    )(M, xr)
    return jnp.transpose(out.reshape(Vo, B, C), (1, 2, 0))


# speedup vs baseline: 1.1048x; 1.1048x over previous
import jax
import jax.numpy as jnp
from jax.experimental import pallas as pl

_BM = 256


def _mm_kernel(x_ref, m_ref, o_ref):
    xv = x_ref[...].astype(jnp.bfloat16)
    m = m_ref[...].astype(jnp.bfloat16)
    o_ref[...] = jax.lax.dot_general(
        xv, m, (((1,), (1,)), ((), ())),
        preferred_element_type=jnp.float32)


def kernel(x, M):
    B, C, V = x.shape
    Vo = M.shape[0]
    N = B * C
    x_flat = x.reshape(N, V)
    out_t = pl.pallas_call(
        _mm_kernel,
        grid=(pl.cdiv(Vo, _BM),),
        in_specs=[
            pl.BlockSpec((N, V), lambda i: (0, 0)),
            pl.BlockSpec((_BM, V), lambda i: (i, 0)),
        ],
        out_specs=pl.BlockSpec((N, _BM), lambda i: (0, i)),
        out_shape=jax.ShapeDtypeStruct((N, Vo), jnp.float32),
    )(x_flat, M)
    return out_t.reshape(B, C, Vo)


# speedup vs baseline: 1.1053x; 1.0005x over previous
import jax
import jax.numpy as jnp
from jax.experimental import pallas as pl

_BM = 128


def _mm_kernel(x_ref, m_ref, o_ref):
    xv = x_ref[...].astype(jnp.bfloat16)
    m = m_ref[...].astype(jnp.bfloat16)
    o_ref[...] = jax.lax.dot_general(
        xv, m, (((1,), (1,)), ((), ())),
        preferred_element_type=jnp.float32)


def kernel(x, M):
    B, C, V = x.shape
    Vo = M.shape[0]
    N = B * C
    x_flat = x.reshape(N, V)
    out_t = pl.pallas_call(
        _mm_kernel,
        grid=(pl.cdiv(Vo, _BM),),
        in_specs=[
            pl.BlockSpec((N, V), lambda i: (0, 0)),
            pl.BlockSpec((_BM, V), lambda i: (i, 0)),
        ],
        out_specs=pl.BlockSpec((N, _BM), lambda i: (0, i)),
        out_shape=jax.ShapeDtypeStruct((N, Vo), jnp.float32),
    )(x_flat, M)
    return out_t.reshape(B, C, Vo)
